# Initial kernel scaffold; baseline (speedup 1.0000x reference)
import jax
import jax.numpy as jnp
from jax.experimental import pallas as pl


def kernel(x, emb_table):
    raise NotImplementedError("write your pallas kernel here")


# speedup vs baseline: 7.0037x; 7.0037x over previous
import jax
import jax.numpy as jnp
from jax.experimental import pallas as pl


def _copy_body(emb_ref, out_ref):
    out_ref[...] = emb_ref[...]


def kernel(x, emb_table):
    seq_len = x.shape[1]
    hidden = emb_table.shape[1]
    out = pl.pallas_call(
        _copy_body,
        grid=(1,),
        in_specs=[pl.BlockSpec((seq_len, hidden), lambda i: (0, 0))],
        out_specs=pl.BlockSpec((seq_len, hidden), lambda i: (0, 0)),
        out_shape=jax.ShapeDtypeStruct((seq_len, hidden), emb_table.dtype),
    )(emb_table)
    return out[None]
